# Initial kernel scaffold; baseline (speedup 1.0000x reference)
import jax
import jax.numpy as jnp
from jax.experimental import pallas as pl


def kernel(x, embedding_table):
    raise NotImplementedError("write your pallas kernel here")


# speedup vs baseline: 2.8280x; 2.8280x over previous
import jax
import jax.numpy as jnp
from jax.experimental import pallas as pl


_SEQ_BLOCK = 512


def _add_block(x_ref, tab_ref, o_ref):
    o_ref[...] = x_ref[...] + tab_ref[...]


def kernel(x, embedding_table):
    batch, seq_len, d_model = x.shape
    table = embedding_table[:seq_len]

    blk = _SEQ_BLOCK
    if seq_len % blk != 0:
        blk = seq_len
    grid = (seq_len // blk, batch)

    return pl.pallas_call(
        _add_block,
        grid=grid,
        in_specs=[
            pl.BlockSpec((1, blk, d_model), lambda i, b: (b, i, 0)),
            pl.BlockSpec((blk, d_model), lambda i, b: (i, 0)),
        ],
        out_specs=pl.BlockSpec((1, blk, d_model), lambda i, b: (b, i, 0)),
        out_shape=jax.ShapeDtypeStruct((batch, seq_len, d_model), x.dtype),
    )(x, table)


# speedup vs baseline: 3.1673x; 1.1200x over previous
import jax
import jax.numpy as jnp
from jax.experimental import pallas as pl


_SEQ_BLOCK = 1024


def _add_block(x_ref, tab_ref, o_ref):
    o_ref[...] = x_ref[...] + tab_ref[...]


def kernel(x, embedding_table):
    batch, seq_len, d_model = x.shape
    table = embedding_table[:seq_len]

    blk = _SEQ_BLOCK
    if seq_len % blk != 0:
        blk = seq_len
    grid = (seq_len // blk, batch)

    return pl.pallas_call(
        _add_block,
        grid=grid,
        in_specs=[
            pl.BlockSpec((1, blk, d_model), lambda i, b: (b, i, 0)),
            pl.BlockSpec((blk, d_model), lambda i, b: (i, 0)),
        ],
        out_specs=pl.BlockSpec((1, blk, d_model), lambda i, b: (b, i, 0)),
        out_shape=jax.ShapeDtypeStruct((batch, seq_len, d_model), x.dtype),
    )(x, table)


# speedup vs baseline: 3.3060x; 1.0438x over previous
import jax
import jax.numpy as jnp
from jax.experimental import pallas as pl


_SEQ_BLOCK = 2048


def _add_block(x_ref, tab_ref, o_ref):
    o_ref[...] = x_ref[...] + tab_ref[...]


def kernel(x, embedding_table):
    batch, seq_len, d_model = x.shape
    table = embedding_table[:seq_len]

    blk = _SEQ_BLOCK
    if seq_len % blk != 0:
        blk = seq_len
    grid = (seq_len // blk, batch)

    return pl.pallas_call(
        _add_block,
        grid=grid,
        in_specs=[
            pl.BlockSpec((1, blk, d_model), lambda i, b: (b, i, 0)),
            pl.BlockSpec((blk, d_model), lambda i, b: (i, 0)),
        ],
        out_specs=pl.BlockSpec((1, blk, d_model), lambda i, b: (b, i, 0)),
        out_shape=jax.ShapeDtypeStruct((batch, seq_len, d_model), x.dtype),
    )(x, table)
